# baseline (speedup 1.0000x reference)
import functools

import numpy as np

import jax
import jax.numpy as jnp
from jax import lax
from jax.experimental import pallas as pl
from jax.experimental.pallas import tpu as pltpu
from jax.experimental.pallas import tpu_sc as plsc

N = 10000
E = 320000
D = 128
DE = 16

NC = 2
NS = 16
NW = NC * NS
L = 16

C = 40
K = E // (NW * C)
E_W = E // NW
NP = 10240

_PERM = np.concatenate(
    [np.arange(32 * j, 32 * j + 32).reshape(16, 2).T.reshape(32)
     for j in range(4)])
_UNPERM_MAT = np.zeros((D, D), np.float32)
_UNPERM_MAT[np.arange(D), _PERM] = 1.0

_mesh = plsc.VectorSubcoreMesh(
    core_axis_name="c", subcore_axis_name="s", num_cores=NC, num_subcores=NS)


def _mm_node_body(x_ref, wa_ref, wb_ref, oa_ref, ob_ref):
    xb = x_ref[...]
    oa_ref[...] = jnp.dot(xb, wa_ref[...], preferred_element_type=jnp.float32)
    ob_ref[...] = jnp.dot(xb, wb_ref[...], preferred_element_type=jnp.float32)


def _mm_node(x, wa, wb):
    blk = 1000
    return pl.pallas_call(
        _mm_node_body,
        grid=(N // blk,),
        in_specs=[
            pl.BlockSpec((blk, D), lambda i: (i, 0)),
            pl.BlockSpec((D, D), lambda i: (0, 0)),
            pl.BlockSpec((D, D), lambda i: (0, 0)),
        ],
        out_specs=[
            pl.BlockSpec((blk, D), lambda i: (i, 0)),
            pl.BlockSpec((blk, D), lambda i: (i, 0)),
        ],
        out_shape=[
            jax.ShapeDtypeStruct((N, D), jnp.float32),
            jax.ShapeDtypeStruct((N, D), jnp.float32),
        ],
    )(x, wa, wb)


def _mm_edge_body(x_ref, w_ref, o_ref):
    o_ref[...] = jnp.dot(x_ref[...], w_ref[...],
                         preferred_element_type=jnp.float32
                         ).astype(jnp.bfloat16)


def _mm_edge(ea, w0):
    blk = 2000
    return pl.pallas_call(
        _mm_edge_body,
        grid=(E // blk,),
        in_specs=[
            pl.BlockSpec((blk, DE), lambda i: (i, 0)),
            pl.BlockSpec((DE, D), lambda i: (0, 0)),
        ],
        out_specs=pl.BlockSpec((blk, D), lambda i: (i, 0)),
        out_shape=jax.ShapeDtypeStruct((E, D), jnp.bfloat16),
    )(ea, w0)


def _norm_mm_body(apply_silu, matmul, acc_ref, den_ref, b_ref, wa_ref, wb_ref,
                  oa_ref, ob_ref):
    db = den_ref[0]
    inv = 1.0 / (db[0] + db[1] + 1e-16)
    h = (acc_ref[0] + acc_ref[1]) * inv[:, None] + b_ref[...]
    if apply_silu:
        h = h * jax.nn.sigmoid(h)
    if matmul:
        oa_ref[...] = jnp.dot(h, wa_ref[...],
                              preferred_element_type=jnp.float32)
        ob_ref[...] = jnp.dot(h, wb_ref[...],
                              preferred_element_type=jnp.float32)
    else:
        oa_ref[...] = jnp.dot(h, wa_ref[...],
                              preferred_element_type=jnp.float32)


def _norm_mm(acc, den_r, b, wa, wb):
    blk = 1000
    return pl.pallas_call(
        functools.partial(_norm_mm_body, True, True),
        grid=(N // blk,),
        in_specs=[
            pl.BlockSpec((NC, blk, D), lambda i: (0, i, 0)),
            pl.BlockSpec((1, NC, blk), lambda i: (i, 0, 0)),
            pl.BlockSpec((D,), lambda i: (0,)),
            pl.BlockSpec((D, D), lambda i: (0, 0)),
            pl.BlockSpec((D, D), lambda i: (0, 0)),
        ],
        out_specs=[
            pl.BlockSpec((blk, D), lambda i: (i, 0)),
            pl.BlockSpec((blk, D), lambda i: (i, 0)),
        ],
        out_shape=[
            jax.ShapeDtypeStruct((N, D), jnp.float32),
            jax.ShapeDtypeStruct((N, D), jnp.float32),
        ],
    )(acc, den_r, b, wa, wb)


def _norm_out_body(acc_ref, den_ref, b_ref, pm_ref, o_ref):
    _norm_mm_body(False, False, acc_ref, den_ref, b_ref, pm_ref, None,
                  o_ref, None)


def _norm_out(acc, den_r, b, pm):
    blk = 1000
    return pl.pallas_call(
        _norm_out_body,
        grid=(N // blk,),
        in_specs=[
            pl.BlockSpec((NC, blk, D), lambda i: (0, i, 0)),
            pl.BlockSpec((1, NC, blk), lambda i: (i, 0, 0)),
            pl.BlockSpec((D,), lambda i: (0,)),
            pl.BlockSpec((D, D), lambda i: (0, 0)),
        ],
        out_specs=pl.BlockSpec((blk, D), lambda i: (i, 0)),
        out_shape=jax.ShapeDtypeStruct((N, D), jnp.float32),
    )(acc, den_r, b, pm)


def _edge_body(xl_hbm, xr_hbm, et_hbm, gidx_hbm, didx_hbm, att_hbm,
               acc_out, den_out,
               gi0, gi1, si0, si1, xlb0, xlb1, xrb0, xrb1, etb0, etb1,
               scb0, scb1, dv0, dv1, att_v,
               s_gi0, s_gi1, s_si0, s_si1, s_gl0, s_gl1, s_gr0, s_gr1,
               s_e0, s_e1, s_sc0, s_sc1, s_dn0, s_dn1,
               acc_sh, den_sh):
    cid = lax.axis_index("c")
    sid = lax.axis_index("s")
    w = cid * NS + sid
    gi = (gi0, gi1)
    si = (si0, si1)
    xlb = (xlb0, xlb1)
    xrb = (xrb0, xrb1)
    etb = (etb0, etb1)
    scb = (scb0, scb1)
    dvb = (dv0, dv1)
    s_gi = (s_gi0, s_gi1)
    s_si = (s_si0, s_si1)
    s_gl = (s_gl0, s_gl1)
    s_gr = (s_gr0, s_gr1)
    s_e = (s_e0, s_e1)
    s_sc = (s_sc0, s_sc1)
    s_dn = (s_dn0, s_dn1)

    def _zero_sc(i, _):
        for j in range(D // L):
            scb0[i, pl.ds(L * j, L)] = jnp.zeros((L,), jnp.float32)
        return 0
    lax.fori_loop(0, C, _zero_sc, 0)
    for i in range(48 // L):
        dv0[pl.ds(L * i, L)] = jnp.zeros((L,), jnp.float32)
    zcps = []
    for p in range(16):
        r0 = sid * 640 + p * C
        zcps.append(pltpu.async_copy(scb0, acc_sh.at[pl.ds(r0, C)], s_sc0))
        zcps.append(pltpu.async_copy(dv0.at[pl.ds(0, C)],
                                     den_sh.at[pl.ds(r0, C)], s_dn0))
    for cp in zcps:
        cp.wait()
    plsc.subcore_barrier()

    pltpu.sync_copy(att_hbm, att_v)
    att_regs = [att_v[pl.ds(L * j, L)] for j in range(D // L)]
    lane_iota = lax.iota(jnp.int32, L)

    def _issue_gathers(k, q):
        pltpu.async_copy(xr_hbm.at[gi[q].at[pl.ds(0, C)]], xrb[q], s_gr[q])
        pltpu.async_copy(xl_hbm.at[gi[q].at[pl.ds(C, C)]], xlb[q], s_gl[q])
        pltpu.async_copy(
            et_hbm.at[pl.ds(pl.multiple_of(
                w * (E_W // 2) + k * (C // 2) - 4 * q, 8), 24)],
            etb[q], s_e[q])

    def _edges(p, rows, row0):
        evec = jnp.zeros((L,), jnp.float32)
        erow0 = 4 * p + row0 // 2
        for e in range(rows):
            row = row0 + e
            acc = jnp.zeros((L,), jnp.float32)
            xl_regs = []
            for jj in range(D // (2 * L)):
                ete, eto = plsc.unpack(
                    plsc.bitcast(
                        etb[p][erow0 + e // 2,
                               pl.ds((D // 2) * (e % 2) + L * jj, L)],
                        jnp.bfloat16),
                    format=plsc.PackFormat.INTERLEAVED)
                xre = xrb[p][row, pl.ds(2 * L * jj, L)]
                xro = xrb[p][row, pl.ds(2 * L * jj + L, L)]
                a0 = xlb[p][row, pl.ds(2 * L * jj, L)]
                a1 = xlb[p][row, pl.ds(2 * L * jj + L, L)]
                xl_regs.append(a0)
                xl_regs.append(a1)
                m0 = a0 + xre + ete
                m1 = a1 + xro + eto
                acc = acc + jnp.maximum(m0, 0.2 * m0) * att_regs[2 * jj]
                acc = acc + jnp.maximum(m1, 0.2 * m1) * att_regs[2 * jj + 1]
            exv = jnp.exp(jnp.zeros((L,), jnp.float32) + jnp.sum(acc))
            evec = jnp.where(lane_iota == e, exv, evec)
            for j in range(D // L):
                scb[p][row, pl.ds(L * j, L)] = xl_regs[j] * exv
        return evec

    def _compute(p):
        def _group(g, _):
            dvb[p][pl.ds(g * L, L)] = _edges(p, L, g * L)
            return 0
        full = C // L
        lax.fori_loop(0, full, _group, 0)
        if C - full * L:
            dvb[p][pl.ds(full * L, L)] = _edges(p, C - full * L, full * L)

    pltpu.sync_copy(gidx_hbm.at[pl.ds(w * 2 * E_W, 2 * C)], gi[0])
    _issue_gathers(0, 0)
    pltpu.async_copy(gidx_hbm.at[pl.ds(w * 2 * E_W + 2 * C, 2 * C)],
                     gi[1], s_gi[1])

    def _pair(i, _):
        for p in (0, 1):
            k = 2 * i + p
            q = 1 - p
            pltpu.make_async_copy(
                xr_hbm.at[gi[p].at[pl.ds(0, C)]], xrb[p], s_gr[p]).wait()
            pltpu.make_async_copy(
                xl_hbm.at[gi[p].at[pl.ds(C, C)]], xlb[p], s_gl[p]).wait()
            pltpu.make_async_copy(
                et_hbm.at[pl.ds(pl.multiple_of(
                    w * (E_W // 2) + k * (C // 2) - 4 * p, 8), 24)],
                etb[p], s_e[p]).wait()

            @pl.when(k >= 2)
            def _():
                pltpu.make_async_copy(
                    scb[p], acc_sh.at[si[p]], s_sc[p]).wait()
                pltpu.make_async_copy(
                    dvb[p].at[pl.ds(0, C)], den_sh.at[si[p]], s_dn[p]).wait()

            pltpu.async_copy(didx_hbm.at[pl.ds(w * E_W + k * C, C)],
                             si[p], s_si[p])

            @pl.when(k <= K - 3)
            def _():
                pltpu.async_copy(
                    gidx_hbm.at[pl.ds(w * 2 * E_W + (k + 2) * 2 * C, 2 * C)],
                    gi[p], s_gi[p])

            @pl.when(k <= K - 2)
            def _():
                pltpu.make_async_copy(
                    gidx_hbm.at[pl.ds(w * 2 * E_W, 2 * C)], gi[q],
                    s_gi[q]).wait()
                _issue_gathers(k + 1, q)

            _compute(p)

            pltpu.make_async_copy(
                didx_hbm.at[pl.ds(w * E_W + k * C, C)], si[p], s_si[p]).wait()
            pltpu.async_copy(scb[p], acc_sh.at[si[p]], s_sc[p], add=True)
            pltpu.async_copy(dvb[p].at[pl.ds(0, C)], den_sh.at[si[p]],
                             s_dn[p], add=True)
        return 0

    lax.fori_loop(0, K // 2, _pair, 0)
    for p in (0, 1):
        pltpu.make_async_copy(scb[p], acc_sh.at[si[p]], s_sc[p]).wait()
        pltpu.make_async_copy(
            dvb[p].at[pl.ds(0, C)], den_sh.at[si[p]], s_dn[p]).wait()
    plsc.subcore_barrier()

    for p in range(5):
        r0 = sid * 640 + p * 128
        pltpu.sync_copy(acc_sh.at[pl.ds(r0, 128)],
                        acc_out.at[cid, pl.ds(r0, 128)])
    d0 = sid * (NP // NS)
    pltpu.sync_copy(den_sh.at[pl.ds(d0, NP // NS)],
                    den_out.at[cid, pl.ds(d0, NP // NS)])


_edge_kernel = pl.kernel(
    _edge_body,
    out_type=[
        jax.ShapeDtypeStruct((NC, NP, D), jnp.float32),
        jax.ShapeDtypeStruct((NC, NP), jnp.float32),
    ],
    mesh=_mesh,
    compiler_params=pltpu.CompilerParams(needs_layout_passes=False),
    scratch_types=[
        pltpu.VMEM((2 * C,), jnp.int32),
        pltpu.VMEM((2 * C,), jnp.int32),
        pltpu.VMEM((C,), jnp.int32),
        pltpu.VMEM((C,), jnp.int32),
        pltpu.VMEM((C, D), jnp.float32),
        pltpu.VMEM((C, D), jnp.float32),
        pltpu.VMEM((C, D), jnp.float32),
        pltpu.VMEM((C, D), jnp.float32),
        pltpu.VMEM((24, D), jnp.int32),
        pltpu.VMEM((24, D), jnp.int32),
        pltpu.VMEM((C, D), jnp.float32),
        pltpu.VMEM((C, D), jnp.float32),
        pltpu.VMEM((48,), jnp.float32),
        pltpu.VMEM((48,), jnp.float32),
        pltpu.VMEM((D,), jnp.float32),
        pltpu.SemaphoreType.DMA,
        pltpu.SemaphoreType.DMA,
        pltpu.SemaphoreType.DMA,
        pltpu.SemaphoreType.DMA,
        pltpu.SemaphoreType.DMA,
        pltpu.SemaphoreType.DMA,
        pltpu.SemaphoreType.DMA,
        pltpu.SemaphoreType.DMA,
        pltpu.SemaphoreType.DMA,
        pltpu.SemaphoreType.DMA,
        pltpu.SemaphoreType.DMA,
        pltpu.SemaphoreType.DMA,
        pltpu.SemaphoreType.DMA,
        pltpu.SemaphoreType.DMA,
        pltpu.VMEM_SHARED((NP, D), jnp.float32),
        pltpu.VMEM_SHARED((NP,), jnp.float32),
    ],
)


def kernel(x, edge_index, edge_attr, Wl0, Wr0, We0, att0, b0,
           Wl1, Wr1, We1, att1, b1):
    s3 = edge_index[0].reshape(NW, K, 1, C)
    d3 = edge_index[1].reshape(NW, K, 1, C)
    gidx = jnp.concatenate([d3, s3], axis=2).reshape(-1)
    didx = d3.reshape(-1)

    perm = jnp.asarray(_PERM)
    pm = jnp.asarray(_UNPERM_MAT)

    xl0, xr0 = _mm_node(x, Wl0[:, perm], Wr0[:, perm])
    et0 = _mm_edge(edge_attr, We0)
    et1 = _mm_edge(edge_attr, We1)

    et0w = jax.lax.bitcast_convert_type(
        et0.reshape(E, D // 2, 2), jnp.int32).reshape(E // 2, D)
    acc0, den0 = _edge_kernel(xl0, xr0, et0w, gidx, didx, att0[perm])
    den0_r = den0[:, :N].reshape(NC, N // 1000, 1000).transpose(1, 0, 2)
    xl1, xr1 = _norm_mm(acc0, den0_r, b0[perm], Wl1[perm][:, perm],
                        Wr1[perm][:, perm])

    et1w = jax.lax.bitcast_convert_type(
        et1.reshape(E, D // 2, 2), jnp.int32).reshape(E // 2, D)
    acc1, den1 = _edge_kernel(xl1, xr1, et1w, gidx, didx, att1[perm])
    den1_r = den1[:, :N].reshape(NC, N // 1000, 1000).transpose(1, 0, 2)
    return _norm_out(acc1, den1_r, b1[perm], pm)


# speedup vs baseline: 2.6115x; 2.6115x over previous
import functools

import jax
import jax.numpy as jnp
from jax import lax
from jax.experimental import pallas as pl
from jax.experimental.pallas import tpu as pltpu
from jax.experimental.pallas import tpu_sc as plsc

N = 10000
E = 320000
D = 128
DE = 16

NC = 2
NS = 16
NW = NC * NS
L = 16

C = 40
K = E // (NW * C)
E_W = E // NW
NP = 10240

_mesh = plsc.VectorSubcoreMesh(
    core_axis_name="c", subcore_axis_name="s", num_cores=NC, num_subcores=NS)


def _mm_node_body(x_ref, wa_ref, wb_ref, oa_ref, ob_ref):
    xb = x_ref[...]
    oa_ref[...] = jnp.dot(xb, wa_ref[...], preferred_element_type=jnp.float32)
    ob_ref[...] = jnp.dot(xb, wb_ref[...], preferred_element_type=jnp.float32)


def _mm_node(x, wa, wb):
    blk = 1000
    return pl.pallas_call(
        _mm_node_body,
        grid=(N // blk,),
        in_specs=[
            pl.BlockSpec((blk, D), lambda i: (i, 0)),
            pl.BlockSpec((D, D), lambda i: (0, 0)),
            pl.BlockSpec((D, D), lambda i: (0, 0)),
        ],
        out_specs=[
            pl.BlockSpec((blk, D), lambda i: (i, 0)),
            pl.BlockSpec((blk, D), lambda i: (i, 0)),
        ],
        out_shape=[
            jax.ShapeDtypeStruct((N, D), jnp.float32),
            jax.ShapeDtypeStruct((N, D), jnp.float32),
        ],
    )(x, wa, wb)


def _mm_edge_body(x_ref, w_ref, o_ref):
    o_ref[...] = jnp.dot(x_ref[...], w_ref[...],
                         preferred_element_type=jnp.float32)


def _mm_edge(ea, w0):
    blk = 2000
    return pl.pallas_call(
        _mm_edge_body,
        grid=(E // blk,),
        in_specs=[
            pl.BlockSpec((blk, DE), lambda i: (i, 0)),
            pl.BlockSpec((DE, D), lambda i: (0, 0)),
        ],
        out_specs=pl.BlockSpec((blk, D), lambda i: (i, 0)),
        out_shape=jax.ShapeDtypeStruct((E, D), jnp.float32),
    )(ea, w0)


def _norm_mm_body(apply_silu, matmul, acc_ref, den_ref, b_ref, wa_ref, wb_ref,
                  oa_ref, ob_ref):
    db = den_ref[0]
    inv = 1.0 / (db[0] + db[1] + 1e-16)
    h = (acc_ref[0] + acc_ref[1]) * inv[:, None] + b_ref[...]
    if apply_silu:
        h = h * jax.nn.sigmoid(h)
    if matmul:
        oa_ref[...] = jnp.dot(h, wa_ref[...],
                              preferred_element_type=jnp.float32)
        ob_ref[...] = jnp.dot(h, wb_ref[...],
                              preferred_element_type=jnp.float32)
    else:
        oa_ref[...] = h


def _norm_mm(acc, den_r, b, wa, wb):
    blk = 1000
    return pl.pallas_call(
        functools.partial(_norm_mm_body, True, True),
        grid=(N // blk,),
        in_specs=[
            pl.BlockSpec((NC, blk, D), lambda i: (0, i, 0)),
            pl.BlockSpec((1, NC, blk), lambda i: (i, 0, 0)),
            pl.BlockSpec((D,), lambda i: (0,)),
            pl.BlockSpec((D, D), lambda i: (0, 0)),
            pl.BlockSpec((D, D), lambda i: (0, 0)),
        ],
        out_specs=[
            pl.BlockSpec((blk, D), lambda i: (i, 0)),
            pl.BlockSpec((blk, D), lambda i: (i, 0)),
        ],
        out_shape=[
            jax.ShapeDtypeStruct((N, D), jnp.float32),
            jax.ShapeDtypeStruct((N, D), jnp.float32),
        ],
    )(acc, den_r, b, wa, wb)


def _norm_out_body(acc_ref, den_ref, b_ref, o_ref):
    _norm_mm_body(False, False, acc_ref, den_ref, b_ref, None, None,
                  o_ref, None)


def _norm_out(acc, den_r, b):
    blk = 1000
    return pl.pallas_call(
        _norm_out_body,
        grid=(N // blk,),
        in_specs=[
            pl.BlockSpec((NC, blk, D), lambda i: (0, i, 0)),
            pl.BlockSpec((1, NC, blk), lambda i: (i, 0, 0)),
            pl.BlockSpec((D,), lambda i: (0,)),
        ],
        out_specs=pl.BlockSpec((blk, D), lambda i: (i, 0)),
        out_shape=jax.ShapeDtypeStruct((N, D), jnp.float32),
    )(acc, den_r, b)


def _edge_body(xl_hbm, xr_hbm, et_hbm, gidx_hbm, didx_hbm, att_hbm,
               acc_out, den_out,
               gi0, gi1, si0, si1, xlb0, xlb1, xrb0, xrb1, etb0, etb1,
               scb0, scb1, dv0, dv1, att_v,
               s_gi0, s_gi1, s_si0, s_si1, s_gl0, s_gl1, s_gr0, s_gr1,
               s_e0, s_e1, s_sc0, s_sc1, s_dn0, s_dn1,
               acc_sh, den_sh):
    cid = lax.axis_index("c")
    sid = lax.axis_index("s")
    w = cid * NS + sid
    gi = (gi0, gi1)
    si = (si0, si1)
    xlb = (xlb0, xlb1)
    xrb = (xrb0, xrb1)
    etb = (etb0, etb1)
    scb = (scb0, scb1)
    dvb = (dv0, dv1)
    s_gi = (s_gi0, s_gi1)
    s_si = (s_si0, s_si1)
    s_gl = (s_gl0, s_gl1)
    s_gr = (s_gr0, s_gr1)
    s_e = (s_e0, s_e1)
    s_sc = (s_sc0, s_sc1)
    s_dn = (s_dn0, s_dn1)

    def _zero_sc(i, _):
        for j in range(D // L):
            scb0[i, pl.ds(L * j, L)] = jnp.zeros((L,), jnp.float32)
        return 0
    lax.fori_loop(0, C, _zero_sc, 0)
    for i in range(48 // L):
        dv0[pl.ds(L * i, L)] = jnp.zeros((L,), jnp.float32)
    zcps = []
    for p in range(16):
        r0 = sid * 640 + p * C
        zcps.append(pltpu.async_copy(scb0, acc_sh.at[pl.ds(r0, C)], s_sc0))
        zcps.append(pltpu.async_copy(dv0.at[pl.ds(0, C)],
                                     den_sh.at[pl.ds(r0, C)], s_dn0))
    for cp in zcps:
        cp.wait()
    plsc.subcore_barrier()

    pltpu.sync_copy(att_hbm, att_v)
    att_regs = [att_v[pl.ds(L * j, L)] for j in range(D // L)]
    lane_iota = lax.iota(jnp.int32, L)

    def _issue_gathers(k, q):
        pltpu.async_copy(xr_hbm.at[gi[q].at[pl.ds(0, C)]], xrb[q], s_gr[q])
        pltpu.async_copy(xl_hbm.at[gi[q].at[pl.ds(C, C)]], xlb[q], s_gl[q])
        pltpu.async_copy(et_hbm.at[pl.ds(w * E_W + k * C, C)], etb[q], s_e[q])

    def _edges(p, rows, row0):
        evec = jnp.zeros((L,), jnp.float32)
        for e in range(rows):
            row = row0 + e
            acc = jnp.zeros((L,), jnp.float32)
            xl_regs = []
            for j in range(D // L):
                sl = pl.ds(L * j, L)
                a = xlb[p][row, sl]
                xl_regs.append(a)
                m = a + xrb[p][row, sl] + etb[p][row, sl]
                acc = acc + jnp.maximum(m, 0.2 * m) * att_regs[j]
            exv = jnp.exp(jnp.zeros((L,), jnp.float32) + jnp.sum(acc))
            evec = jnp.where(lane_iota == e, exv, evec)
            for j in range(D // L):
                scb[p][row, pl.ds(L * j, L)] = xl_regs[j] * exv
        return evec

    def _compute(p):
        def _group(g, _):
            dvb[p][pl.ds(g * L, L)] = _edges(p, L, g * L)
            return 0
        full = C // L
        lax.fori_loop(0, full, _group, 0)
        if C - full * L:
            dvb[p][pl.ds(full * L, L)] = _edges(p, C - full * L, full * L)

    pltpu.sync_copy(gidx_hbm.at[pl.ds(w * 2 * E_W, 2 * C)], gi[0])
    _issue_gathers(0, 0)
    pltpu.async_copy(gidx_hbm.at[pl.ds(w * 2 * E_W + 2 * C, 2 * C)],
                     gi[1], s_gi[1])

    def _pair(i, _):
        for p in (0, 1):
            k = 2 * i + p
            q = 1 - p
            pltpu.make_async_copy(
                xr_hbm.at[gi[p].at[pl.ds(0, C)]], xrb[p], s_gr[p]).wait()
            pltpu.make_async_copy(
                xl_hbm.at[gi[p].at[pl.ds(C, C)]], xlb[p], s_gl[p]).wait()
            pltpu.make_async_copy(
                et_hbm.at[pl.ds(w * E_W + k * C, C)], etb[p], s_e[p]).wait()

            @pl.when(k >= 2)
            def _():
                pltpu.make_async_copy(
                    scb[p], acc_sh.at[si[p]], s_sc[p]).wait()
                pltpu.make_async_copy(
                    dvb[p].at[pl.ds(0, C)], den_sh.at[si[p]], s_dn[p]).wait()

            pltpu.async_copy(didx_hbm.at[pl.ds(w * E_W + k * C, C)],
                             si[p], s_si[p])

            @pl.when(k <= K - 3)
            def _():
                pltpu.async_copy(
                    gidx_hbm.at[pl.ds(w * 2 * E_W + (k + 2) * 2 * C, 2 * C)],
                    gi[p], s_gi[p])

            @pl.when(k <= K - 2)
            def _():
                pltpu.make_async_copy(
                    gidx_hbm.at[pl.ds(w * 2 * E_W, 2 * C)], gi[q],
                    s_gi[q]).wait()
                _issue_gathers(k + 1, q)

            _compute(p)

            pltpu.make_async_copy(
                didx_hbm.at[pl.ds(w * E_W + k * C, C)], si[p], s_si[p]).wait()
            pltpu.async_copy(scb[p], acc_sh.at[si[p]], s_sc[p], add=True)
            pltpu.async_copy(dvb[p].at[pl.ds(0, C)], den_sh.at[si[p]],
                             s_dn[p], add=True)
        return 0

    lax.fori_loop(0, K // 2, _pair, 0)
    for p in (0, 1):
        pltpu.make_async_copy(scb[p], acc_sh.at[si[p]], s_sc[p]).wait()
        pltpu.make_async_copy(
            dvb[p].at[pl.ds(0, C)], den_sh.at[si[p]], s_dn[p]).wait()
    plsc.subcore_barrier()

    for p in range(5):
        r0 = sid * 640 + p * 128
        pltpu.sync_copy(acc_sh.at[pl.ds(r0, 128)],
                        acc_out.at[cid, pl.ds(r0, 128)])
    d0 = sid * (NP // NS)
    pltpu.sync_copy(den_sh.at[pl.ds(d0, NP // NS)],
                    den_out.at[cid, pl.ds(d0, NP // NS)])


_edge_kernel = pl.kernel(
    _edge_body,
    out_type=[
        jax.ShapeDtypeStruct((NC, NP, D), jnp.float32),
        jax.ShapeDtypeStruct((NC, NP), jnp.float32),
    ],
    mesh=_mesh,
    compiler_params=pltpu.CompilerParams(needs_layout_passes=False),
    scratch_types=[
        pltpu.VMEM((2 * C,), jnp.int32),
        pltpu.VMEM((2 * C,), jnp.int32),
        pltpu.VMEM((C,), jnp.int32),
        pltpu.VMEM((C,), jnp.int32),
        pltpu.VMEM((C, D), jnp.float32),
        pltpu.VMEM((C, D), jnp.float32),
        pltpu.VMEM((C, D), jnp.float32),
        pltpu.VMEM((C, D), jnp.float32),
        pltpu.VMEM((C, D), jnp.float32),
        pltpu.VMEM((C, D), jnp.float32),
        pltpu.VMEM((C, D), jnp.float32),
        pltpu.VMEM((C, D), jnp.float32),
        pltpu.VMEM((48,), jnp.float32),
        pltpu.VMEM((48,), jnp.float32),
        pltpu.VMEM((D,), jnp.float32),
        pltpu.SemaphoreType.DMA,
        pltpu.SemaphoreType.DMA,
        pltpu.SemaphoreType.DMA,
        pltpu.SemaphoreType.DMA,
        pltpu.SemaphoreType.DMA,
        pltpu.SemaphoreType.DMA,
        pltpu.SemaphoreType.DMA,
        pltpu.SemaphoreType.DMA,
        pltpu.SemaphoreType.DMA,
        pltpu.SemaphoreType.DMA,
        pltpu.SemaphoreType.DMA,
        pltpu.SemaphoreType.DMA,
        pltpu.SemaphoreType.DMA,
        pltpu.SemaphoreType.DMA,
        pltpu.VMEM_SHARED((NP, D), jnp.float32),
        pltpu.VMEM_SHARED((NP,), jnp.float32),
    ],
)


def kernel(x, edge_index, edge_attr, Wl0, Wr0, We0, att0, b0,
           Wl1, Wr1, We1, att1, b1):
    s3 = edge_index[0].reshape(NW, K, 1, C)
    d3 = edge_index[1].reshape(NW, K, 1, C)
    gidx = jnp.concatenate([d3, s3], axis=2).reshape(-1)
    didx = d3.reshape(-1)

    xl0, xr0 = _mm_node(x, Wl0, Wr0)
    et0 = _mm_edge(edge_attr, We0)
    et1 = _mm_edge(edge_attr, We1)

    acc0, den0 = _edge_kernel(xl0, xr0, et0, gidx, didx, att0)
    den0_r = den0[:, :N].reshape(NC, N // 1000, 1000).transpose(1, 0, 2)
    xl1, xr1 = _norm_mm(acc0, den0_r, b0, Wl1, Wr1)

    acc1, den1 = _edge_kernel(xl1, xr1, et1, gidx, didx, att1)
    den1_r = den1[:, :N].reshape(NC, N // 1000, 1000).transpose(1, 0, 2)
    return _norm_out(acc1, den1_r, b1)
